# baseline (speedup 1.0000x reference)
import functools

import jax
import jax.numpy as jnp
from jax import lax
from jax.experimental import pallas as pl
from jax.experimental.pallas import tpu as pltpu
from jax.experimental.pallas import tpu_sc as plsc


def _proj_body(TB, E, B, L, x_ref, w_ref, b_ref, n_ref, proj_ref, sumh_ref):
    i = pl.program_id(0)
    x2 = x_ref[...].reshape(TB, L)
    val = jnp.dot(x2, w_ref[...], preferred_element_type=jnp.float32)
    val = val + b_ref[...]
    ii = jax.lax.broadcasted_iota(jnp.int32, (TB, 128), 1)
    NR = B // 128
    srow = jax.lax.broadcasted_iota(jnp.int32, (TB, NR), 1)
    trow = jax.lax.broadcasted_iota(jnp.int32, (TB, NR), 0)
    A = (srow == i * (TB // 128) + trow // 128).astype(jnp.float32)
    Y = jnp.dot(A, n_ref[...], preferred_element_type=jnp.float32)
    tmod = jax.lax.broadcasted_iota(jnp.int32, (TB, 128), 0) % 128
    noise_col = jnp.sum(jnp.where(ii == tmod, Y, 0.0), axis=1, keepdims=True)
    val = val + jnp.where(ii < E, noise_col, 0.0)
    proj_ref[...] = val[:, :3 * E]
    sumh_p = jnp.sum(jnp.where(ii < E, val, 0.0), axis=0, keepdims=True)

    @pl.when(i == 0)
    def _():
        sumh_ref[...] = sumh_p

    @pl.when(i > 0)
    def _():
        sumh_ref[...] = sumh_ref[...] + sumh_p


def _sc_routing(E, TPW, proj, outb, dispb, cntb, projv, outv, dispv, cntv):
    c = lax.axis_index("c")
    s = lax.axis_index("s")
    wid = s * 2 + c
    base = wid * TPW
    pltpu.sync_copy(proj.at[pl.ds(base, TPW)], projv)
    zf = jnp.zeros((16,), jnp.float32)
    for i in range(TPW * E // 16):
        dispv[pl.ds(i * 16, 16)] = zf
    lane = lax.iota(jnp.int32, 16)
    cntacc = zf
    ones = jnp.ones((16,), jnp.float32)
    for g in range(TPW // 16):
        rows = lane + g * 16
        m = plsc.load_gather(projv, [rows, jnp.zeros((16,), jnp.int32)])
        idx = jnp.zeros((16,), jnp.int32)
        for e in range(1, E):
            ce = plsc.load_gather(projv, [rows, jnp.full((16,), e, jnp.int32)])
            upd = ce > m
            m = jnp.where(upd, ce, m)
            idx = jnp.where(upd, e, idx)
        sel0 = plsc.load_gather(projv, [rows, idx + E])
        sel1 = plsc.load_gather(projv, [rows, idx + 2 * E])
        plsc.store_scatter(outv, [rows * 2], m * sel0)
        plsc.store_scatter(outv, [rows * 2 + 1], m * sel1)
        plsc.store_scatter(dispv, [rows * E + idx], ones)
        for e in range(E):
            pc = plsc.all_reduce_population_count(idx == e)
            cntacc = cntacc + jnp.where(lane == e, pc.astype(jnp.float32), 0.0)
    cntv[...] = cntacc
    pltpu.sync_copy(outv, outb.at[pl.ds(base * 2, TPW * 2)])
    pltpu.sync_copy(dispv, dispb.at[pl.ds(base * E, TPW * E)])
    pltpu.sync_copy(cntv, cntb.at[pl.ds(wid * 16, 16)])


def kernel(x, gw, gb, ew, eb):
    B = x.shape[0]
    L = x.shape[2]
    E = gb.shape[0]
    PD = gw.shape[2]
    P = L // PD
    F = ew.shape[1] // 2
    xflat = x.reshape(B * (L // 128), 128)
    noise = jax.random.uniform(
        jax.random.key(42), (B // 128, 128), dtype=jnp.float32)

    Gt = gw[:, 0, :].T
    Wg = ew[:, :, 0, :].reshape(E, 2, F, PD).sum(axis=2)
    Wt = Wg.transpose(2, 1, 0).reshape(PD, 2 * E)
    cols = jnp.concatenate(
        [Gt, Wt, jnp.zeros((PD, 128 - 3 * E), jnp.float32)], axis=1)
    wcat = jnp.tile(cols, (P, 1))
    bsum = (P * eb.reshape(E, 2, F).sum(axis=-1)).T.reshape(2 * E)
    bias = jnp.concatenate(
        [gb * P, bsum, jnp.zeros((128 - 3 * E,), jnp.float32)])[None, :]

    TB = 512
    grid = (B // TB,)
    proj, sumh = pl.pallas_call(
        functools.partial(_proj_body, TB, E, B, L),
        grid=grid,
        in_specs=[
            pl.BlockSpec((TB * (L // 128), 128), lambda i: (i, 0)),
            pl.BlockSpec((L, 128), lambda i: (0, 0)),
            pl.BlockSpec((1, 128), lambda i: (0, 0)),
            pl.BlockSpec((B // 128, 128), lambda i: (0, 0)),
        ],
        out_specs=[
            pl.BlockSpec((TB, 3 * E), lambda i: (i, 0)),
            pl.BlockSpec((1, 128), lambda i: (0, 0)),
        ],
        out_shape=[
            jax.ShapeDtypeStruct((B, 3 * E), jnp.float32),
            jax.ShapeDtypeStruct((1, 128), jnp.float32),
        ],
    )(xflat, wcat, bias, noise)

    NW = 32
    TPW = B // NW
    mesh = plsc.VectorSubcoreMesh(core_axis_name="c", subcore_axis_name="s")
    sc = functools.partial(
        pl.kernel,
        out_type=[
            jax.ShapeDtypeStruct((B * 2,), jnp.float32),
            jax.ShapeDtypeStruct((B * E,), jnp.float32),
            jax.ShapeDtypeStruct((NW * 16,), jnp.float32),
        ],
        mesh=mesh,
        compiler_params=pltpu.CompilerParams(needs_layout_passes=False),
        scratch_types=[
            pltpu.VMEM((TPW, 3 * E), jnp.float32),
            pltpu.VMEM((TPW * 2,), jnp.float32),
            pltpu.VMEM((TPW * E,), jnp.float32),
            pltpu.VMEM((16,), jnp.float32),
        ],
    )(functools.partial(_sc_routing, E, TPW))
    outf, dispf, cntp = sc(proj)

    out = outf.reshape(B, 2)
    disp = dispf.reshape(B, E)
    counts = cntp.reshape(NW, 16).sum(axis=0)[:E]
    loss = jnp.dot(sumh[0, :E], counts) * (E / (B * B))
    return (out, disp, loss)


# speedup vs baseline: 1.0414x; 1.0414x over previous
import functools

import jax
import jax.numpy as jnp
from jax import lax
from jax.experimental import pallas as pl
from jax.experimental.pallas import tpu as pltpu
from jax.experimental.pallas import tpu_sc as plsc


def _proj_body(TB, E, B, L, x_ref, w_ref, b_ref, n_ref, proj_ref, sumh_ref):
    i = pl.program_id(0)
    x2 = x_ref[...].reshape(TB, L)
    val = jnp.dot(x2, w_ref[...], preferred_element_type=jnp.float32)
    val = val + b_ref[...]
    ii = jax.lax.broadcasted_iota(jnp.int32, (TB, 128), 1)
    NR = B // 128
    srow = jax.lax.broadcasted_iota(jnp.int32, (TB, NR), 1)
    trow = jax.lax.broadcasted_iota(jnp.int32, (TB, NR), 0)
    A = (srow == i * (TB // 128) + trow // 128).astype(jnp.float32)
    Y = jnp.dot(A, n_ref[...], preferred_element_type=jnp.float32)
    tmod = jax.lax.broadcasted_iota(jnp.int32, (TB, 128), 0) % 128
    noise_col = jnp.sum(jnp.where(ii == tmod, Y, 0.0), axis=1, keepdims=True)
    val = val + jnp.where(ii < E, noise_col, 0.0)
    proj_ref[...] = val[:, :3 * E]
    sumh_p = jnp.sum(jnp.where(ii < E, val, 0.0), axis=0, keepdims=True)

    @pl.when(i == 0)
    def _():
        sumh_ref[...] = sumh_p

    @pl.when(i > 0)
    def _():
        sumh_ref[...] = sumh_ref[...] + sumh_p


def _sc_routing(E, TPW, proj, outb, dispb, cntb, projv, outv, dispv, cntv):
    c = lax.axis_index("c")
    s = lax.axis_index("s")
    wid = s * 2 + c
    base = wid * TPW
    pltpu.sync_copy(proj.at[pl.ds(base, TPW)], projv)
    zf = jnp.zeros((16,), jnp.float32)
    for i in range(TPW * E // 16):
        dispv[pl.ds(i * 16, 16)] = zf
    lane = lax.iota(jnp.int32, 16)
    cntacc = zf
    ones = jnp.ones((16,), jnp.float32)
    for g in range(TPW // 16):
        rows = lane + g * 16
        m = plsc.load_gather(projv, [rows, jnp.zeros((16,), jnp.int32)])
        idx = jnp.zeros((16,), jnp.int32)
        for e in range(1, E):
            ce = plsc.load_gather(projv, [rows, jnp.full((16,), e, jnp.int32)])
            upd = ce > m
            m = jnp.where(upd, ce, m)
            idx = jnp.where(upd, e, idx)
        sel0 = plsc.load_gather(projv, [rows, idx + E])
        sel1 = plsc.load_gather(projv, [rows, idx + 2 * E])
        plsc.store_scatter(outv, [rows * 2], m * sel0)
        plsc.store_scatter(outv, [rows * 2 + 1], m * sel1)
        plsc.store_scatter(dispv, [rows * E + idx], ones)
        for e in range(E):
            pc = plsc.all_reduce_population_count(idx == e)
            cntacc = cntacc + jnp.where(lane == e, pc.astype(jnp.float32), 0.0)
    cntv[...] = cntacc
    pltpu.sync_copy(outv, outb.at[pl.ds(base * 2, TPW * 2)])
    pltpu.sync_copy(dispv, dispb.at[pl.ds(base * E, TPW * E)])
    pltpu.sync_copy(cntv, cntb.at[pl.ds(wid * 16, 16)])


def kernel(x, gw, gb, ew, eb):
    B = x.shape[0]
    L = x.shape[2]
    E = gb.shape[0]
    PD = gw.shape[2]
    P = L // PD
    F = ew.shape[1] // 2
    xflat = x.reshape(B * (L // 128), 128)
    noise = jax.random.uniform(
        jax.random.key(42), (B // 128, 128), dtype=jnp.float32)

    Gt = gw[:, 0, :].T
    Wg = ew[:, :, 0, :].reshape(E, 2, F, PD).sum(axis=2)
    Wt = Wg.transpose(2, 1, 0).reshape(PD, 2 * E)
    cols = jnp.concatenate(
        [Gt, Wt, jnp.zeros((PD, 128 - 3 * E), jnp.float32)], axis=1)
    wcat = jnp.tile(cols, (P, 1))
    bsum = (P * eb.reshape(E, 2, F).sum(axis=-1)).T.reshape(2 * E)
    bias = jnp.concatenate(
        [gb * P, bsum, jnp.zeros((128 - 3 * E,), jnp.float32)])[None, :]

    TB = 1024
    grid = (B // TB,)
    proj, sumh = pl.pallas_call(
        functools.partial(_proj_body, TB, E, B, L),
        grid=grid,
        in_specs=[
            pl.BlockSpec((TB * (L // 128), 128), lambda i: (i, 0)),
            pl.BlockSpec((L, 128), lambda i: (0, 0)),
            pl.BlockSpec((1, 128), lambda i: (0, 0)),
            pl.BlockSpec((B // 128, 128), lambda i: (0, 0)),
        ],
        out_specs=[
            pl.BlockSpec((TB, 3 * E), lambda i: (i, 0)),
            pl.BlockSpec((1, 128), lambda i: (0, 0)),
        ],
        out_shape=[
            jax.ShapeDtypeStruct((B, 3 * E), jnp.float32),
            jax.ShapeDtypeStruct((1, 128), jnp.float32),
        ],
    )(xflat, wcat, bias, noise)

    NW = 32
    TPW = B // NW
    mesh = plsc.VectorSubcoreMesh(core_axis_name="c", subcore_axis_name="s")
    sc = functools.partial(
        pl.kernel,
        out_type=[
            jax.ShapeDtypeStruct((B * 2,), jnp.float32),
            jax.ShapeDtypeStruct((B * E,), jnp.float32),
            jax.ShapeDtypeStruct((NW * 16,), jnp.float32),
        ],
        mesh=mesh,
        compiler_params=pltpu.CompilerParams(needs_layout_passes=False),
        scratch_types=[
            pltpu.VMEM((TPW, 3 * E), jnp.float32),
            pltpu.VMEM((TPW * 2,), jnp.float32),
            pltpu.VMEM((TPW * E,), jnp.float32),
            pltpu.VMEM((16,), jnp.float32),
        ],
    )(functools.partial(_sc_routing, E, TPW))
    outf, dispf, cntp = sc(proj)

    out = outf.reshape(B, 2)
    disp = dispf.reshape(B, E)
    counts = cntp.reshape(NW, 16).sum(axis=0)[:E]
    loss = jnp.dot(sumh[0, :E], counts) * (E / (B * B))
    return (out, disp, loss)
